# baseline (speedup 1.0000x reference)
import functools

import jax
import jax.numpy as jnp
import numpy as np
from jax import lax
from jax.experimental import pallas as pl
from jax.experimental.pallas import tpu as pltpu
from jax.experimental.pallas import tpu_sc as plsc

IMG_H = 64
IMG_W = 64
STRIDE = 16.0
MIN_SIZE = 16.0
PRE_NMS = 6000
POST_NMS = 300
NMS_THRES = 0.7
N_ANCH = 9
NPIX = IMG_H * IMG_W
NBOX = NPIX * N_ANCH
ROWS = NBOX // 128

CROWS = 48
CCAP = CROWS * 128
NMSROWS = 47
SC_W = 18
RT = ROWS // SC_W


def _make_anchor_planes():
    xs = np.arange(0.5, IMG_W + 0.5, 1.0, dtype=np.float32)
    ys = np.arange(0.5, IMG_H + 0.5, 1.0, dtype=np.float32)
    scales = [8.0, 16.0, 32.0]
    ratios = [0.5, 1.0, 2.0]
    hw = np.array([[s * np.sqrt(r), s * (1.0 / np.sqrt(r))] for s in scales for r in ratios],
                  dtype=np.float32)
    Y = np.broadcast_to(ys[None, :, None], (IMG_W, IMG_H, N_ANCH))
    X = np.broadcast_to(xs[:, None, None], (IMG_W, IMG_H, N_ANCH))
    H = np.broadcast_to(hw[None, None, :, 0], (IMG_W, IMG_H, N_ANCH))
    W = np.broadcast_to(hw[None, None, :, 1], (IMG_W, IMG_H, N_ANCH))
    anch = np.stack([Y, X, H, W], axis=-1).reshape(-1, 4).astype(np.float32) * STRIDE
    a = anch.reshape(NPIX, N_ANCH, 4)
    ap = np.zeros((NPIX, 64), np.float32)
    for k in range(4):
        ap[:, 16 * k:16 * k + N_ANCH] = a[:, :, k]
    return ap


_AP = _make_anchor_planes()

_NEG = -3.0


def _trunk_kernel(xp_ref, wk_ref, w2_ref, b2_ref, ap_ref,
                  gam_ref, bet_ref, mu_ref, var_ref,
                  conv1_ref, r_ref):
    i = pl.program_id(0)
    acc = jnp.zeros((512, 512), jnp.float32)
    for dy in range(3):
        for dx in range(3):
            sl = xp_ref[pl.ds(8 * i + dy, 8), dx:dx + 64, :]
            sl = sl.reshape(512, 256)
            acc = acc + jnp.dot(sl, wk_ref[3 * dy + dx],
                                preferred_element_type=jnp.float32)
    c = (acc - mu_ref[...]) * jax.lax.rsqrt(var_ref[...] + 1e-5) * gam_ref[...] + bet_ref[...]
    conv1 = jnp.where(c >= 0, c, 0.1 * c)
    conv1_ref[...] = conv1
    h2 = jnp.dot(conv1, w2_ref[...], preferred_element_type=jnp.float32) + b2_ref[...]
    b_y = h2[:, 0:16]
    b_x = h2[:, 16:32]
    b_h = h2[:, 32:48]
    b_w = h2[:, 48:64]
    c0 = h2[:, 64:80]
    c1 = h2[:, 80:96]
    m = jnp.maximum(c0, c1)
    e0 = jnp.exp(c0 - m)
    e1 = jnp.exp(c1 - m)
    den = e0 + e1
    p0 = e0 / den
    p1 = e1 / den
    a_y = ap_ref[:, 0:16]
    a_x = ap_ref[:, 16:32]
    a_h = ap_ref[:, 32:48]
    a_w = ap_ref[:, 48:64]
    r_y = b_y * a_h + a_y
    r_x = b_x * a_w + a_x
    r_h = jnp.exp(b_h) * a_h
    r_w = jnp.exp(b_w) * a_w
    y1 = r_y - r_h / 2.0
    x1 = r_x - r_w / 2.0
    y2 = r_y + r_h / 2.0
    x2 = r_x + r_w / 2.0
    hi = jnp.float32(IMG_H * STRIDE)
    y1 = jnp.clip(y1, 0.0, hi)
    x1 = jnp.clip(x1, 0.0, hi)
    y2 = jnp.clip(y2, 0.0, hi)
    x2 = jnp.clip(x2, 0.0, hi)
    cy = (y1 + y2) / 2.0
    cx = (x1 + x2) / 2.0
    hh = y2 - y1
    ww = x2 - x1
    valid = (hh >= MIN_SIZE) & (ww >= MIN_SIZE)
    s = jnp.where(valid, p1, -1.0)
    Y1 = cy - hh / 2.0
    X1 = cx - ww / 2.0
    Y2 = cy + hh / 2.0
    X2 = cx + ww / 2.0
    zeros = jnp.zeros((512, 80), jnp.float32)
    r_ref[...] = jnp.concatenate(
        [b_y, b_x, b_h, b_w, p0, p1, Y1, X1, Y2, X2, s, zeros], axis=1)


def _prep_kernel(s_ref, pos_ref, meta_ref):
    s = s_ref[...]
    key = jnp.where(s > -0.5, jax.lax.bitcast_convert_type(s, jnp.int32),
                    jnp.int32(-1))
    nvalid = jnp.sum((key >= 0).astype(jnp.int32))

    def bs_body(_, lohi):
        lo, hi = lohi
        mid = lo + (hi - lo + 1) // 2
        cnt = jnp.sum((key >= mid).astype(jnp.int32))
        take = cnt >= PRE_NMS
        return (jnp.where(take, mid, lo), jnp.where(take, hi, mid - 1))

    lo, _ = jax.lax.fori_loop(0, 32, bs_body,
                              (jnp.int32(0), jnp.int32(0x7F800000)))
    teff = jnp.where(nvalid < PRE_NMS, jnp.int32(0), lo)

    keep = key >= teff
    kf = keep.astype(jnp.float32)

    rl = jax.lax.broadcasted_iota(jnp.int32, (128, 128), 0)
    cl = jax.lax.broadcasted_iota(jnp.int32, (128, 128), 1)
    U = (rl <= cl).astype(jnp.float32)
    p = jnp.dot(kf, U, preferred_element_type=jnp.float32)

    rtot = jnp.broadcast_to(p[:, 127:128], (ROWS, 128))
    ri = jax.lax.broadcasted_iota(jnp.int32, (ROWS, ROWS), 0)
    ci = jax.lax.broadcasted_iota(jnp.int32, (ROWS, ROWS), 1)
    tril = (ci < ri).astype(jnp.float32)
    base = jnp.dot(tril, rtot, preferred_element_type=jnp.float32)

    pos = (base + p - kf).astype(jnp.int32)
    pos = jnp.minimum(pos, jnp.int32(CCAP - 1))
    pos_ref[...] = jnp.where(keep, pos, jnp.int32(CCAP - 1))
    total = jnp.sum(keep.astype(jnp.int32))
    meta_ref[...] = jnp.zeros((1, 128), jnp.int32) + total


def _compact_kernel(pos_h, s_h, y1_h, x1_h, y2_h, x2_h,
                    os_h, oy1_h, ox1_h, oy2_h, ox2_h,
                    pv, sv, y1v, x1v, y2v, x2v, sem):
    w = lax.axis_index("c") * 16 + lax.axis_index("s")

    @pl.when(w < SC_W)
    def _():
        base = w * RT
        pltpu.sync_copy(pos_h.at[pl.ds(base, RT)], pv)
        pltpu.sync_copy(s_h.at[pl.ds(base, RT)], sv)
        pltpu.sync_copy(y1_h.at[pl.ds(base, RT)], y1v)
        pltpu.sync_copy(x1_h.at[pl.ds(base, RT)], x1v)
        pltpu.sync_copy(y2_h.at[pl.ds(base, RT)], y2v)
        pltpu.sync_copy(x2_h.at[pl.ds(base, RT)], x2v)
        cps = []
        for src, dst in ((sv, os_h), (y1v, oy1_h), (x1v, ox1_h),
                         (y2v, oy2_h), (x2v, ox2_h)):
            for r in range(RT):
                cps.append(pltpu.async_copy(src.at[r], dst.at[pv.at[r]], sem))
        for c in cps:
            c.wait()


def _nms_kernel(y1_ref, x1_ref, y2_ref, x2_ref, s_ref, cnt_ref, b0_ref,
                out_ref):
    s = s_ref[...]
    y1 = y1_ref[...]
    x1 = x1_ref[...]
    y2 = y2_ref[...]
    x2 = x2_ref[...]
    area = (y2 - y1) * (x2 - x1)
    lin = (jax.lax.broadcasted_iota(jnp.int32, (NMSROWS, 128), 0) * 128
           + jax.lax.broadcasted_iota(jnp.int32, (NMSROWS, 128), 1))
    cio = jax.lax.broadcasted_iota(jnp.int32, (1, 128), 1)

    total = jnp.sum(cnt_ref[...][:, 0:1])
    active = lin < total
    ms0 = jnp.where(active, s, _NEG)

    def pickrow(ref, r, col):
        row = ref[pl.ds(r, 1), :]
        return jnp.sum(jnp.where(cio == col, row, 0.0))

    have = total > 0
    gmax = jnp.max(ms0)
    gidx = jnp.min(jnp.where(ms0 == gmax, lin, jnp.int32(2 ** 30)))
    b0y1 = jnp.sum(jnp.where(cio == 0, b0_ref[0:1, :], 0.0))
    b0x1 = jnp.sum(jnp.where(cio == 0, b0_ref[1:2, :], 0.0))
    b0y2 = jnp.sum(jnp.where(cio == 0, b0_ref[2:3, :], 0.0))
    b0x2 = jnp.sum(jnp.where(cio == 0, b0_ref[3:4, :], 0.0))

    def body(k, ms):
        mmax = jnp.max(ms)
        empty = mmax < -2.0
        idx0 = jnp.min(jnp.where(ms == mmax, lin, jnp.int32(2 ** 30)))
        idx = jnp.where(empty, gidx, idx0)
        r = idx // 128
        col = idx % 128
        y1i = jnp.where(have, pickrow(y1_ref, r, col), b0y1)
        x1i = jnp.where(have, pickrow(x1_ref, r, col), b0x1)
        y2i = jnp.where(have, pickrow(y2_ref, r, col), b0y2)
        x2i = jnp.where(have, pickrow(x2_ref, r, col), b0x2)
        ai = (y2i - y1i) * (x2i - x1i)
        yy1 = jnp.maximum(y1i, y1)
        xx1 = jnp.maximum(x1i, x1)
        yy2 = jnp.minimum(y2i, y2)
        xx2 = jnp.minimum(x2i, x2)
        inter = jnp.maximum(yy2 - yy1, 0.0) * jnp.maximum(xx2 - xx1, 0.0)
        iou = inter / (ai + area - inter + 1e-9)
        ms = jnp.where((iou <= NMS_THRES) & (lin != idx), ms, _NEG)
        outrow = jnp.where(cio == 0, y1i,
                 jnp.where(cio == 1, x1i,
                 jnp.where(cio == 2, y2i,
                 jnp.where(cio == 3, x2i, 0.0))))
        out_ref[pl.ds(k, 1), :] = outrow
        return ms

    jax.lax.fori_loop(0, POST_NMS, body, ms0)


@functools.partial(jax.jit)
def kernel(x, conv_w, bn_gamma, bn_beta, bn_mean, bn_var,
           rpn_bnd_w, rpn_bnd_b, rpn_class_w, rpn_class_b):
    f32 = jnp.float32
    xh = jnp.transpose(x[0], (1, 2, 0))
    xp = jnp.pad(xh, ((1, 1), (1, 1), (0, 0)))
    wk = jnp.transpose(conv_w, (2, 3, 1, 0)).reshape(9, 256, 512)

    wb4 = jnp.transpose(rpn_bnd_w.reshape(N_ANCH, 4, 512), (1, 0, 2))
    wc2 = jnp.transpose(rpn_class_w.reshape(N_ANCH, 2, 512), (1, 0, 2))

    def col_block(wrows):
        return jnp.pad(jnp.transpose(wrows), ((0, 0), (0, 16 - N_ANCH)))

    w2 = jnp.concatenate([col_block(wb4[0]), col_block(wb4[1]),
                          col_block(wb4[2]), col_block(wb4[3]),
                          col_block(wc2[0]), col_block(wc2[1]),
                          jnp.zeros((512, 32), f32)], axis=1)

    def bias_block(brows):
        return jnp.pad(brows, (0, 16 - N_ANCH))

    bb = jnp.transpose(rpn_bnd_b.reshape(N_ANCH, 4))
    cb = jnp.transpose(rpn_class_b.reshape(N_ANCH, 2))
    b2 = jnp.concatenate([bias_block(bb[0]), bias_block(bb[1]),
                          bias_block(bb[2]), bias_block(bb[3]),
                          bias_block(cb[0]), bias_block(cb[1]),
                          jnp.zeros((32,), f32)])[None, :]

    conv1_flat, R = pl.pallas_call(
        _trunk_kernel,
        grid=(8,),
        in_specs=[
            pl.BlockSpec((66, 66, 256), lambda i: (0, 0, 0)),
            pl.BlockSpec((9, 256, 512), lambda i: (0, 0, 0)),
            pl.BlockSpec((512, 128), lambda i: (0, 0)),
            pl.BlockSpec((1, 128), lambda i: (0, 0)),
            pl.BlockSpec((512, 64), lambda i: (i, 0)),
            pl.BlockSpec((1, 512), lambda i: (0, 0)),
            pl.BlockSpec((1, 512), lambda i: (0, 0)),
            pl.BlockSpec((1, 512), lambda i: (0, 0)),
            pl.BlockSpec((1, 512), lambda i: (0, 0)),
        ],
        out_specs=[
            pl.BlockSpec((512, 512), lambda i: (i, 0)),
            pl.BlockSpec((512, 256), lambda i: (i, 0)),
        ],
        out_shape=[
            jax.ShapeDtypeStruct((NPIX, 512), f32),
            jax.ShapeDtypeStruct((NPIX, 256), f32),
        ],
    )(xp, wk, w2, b2, jnp.asarray(_AP),
      bn_gamma[None, :], bn_beta[None, :], bn_mean[None, :], bn_var[None, :])

    conv1 = jnp.transpose(conv1_flat.reshape(IMG_H, IMG_W, 512), (2, 0, 1))[None]
    rpn_bnds = jnp.stack([R[:, 0:9], R[:, 16:25], R[:, 32:41], R[:, 48:57]],
                         axis=-1).reshape(1, NBOX, 4)
    rpn_class = jnp.stack([R[:, 64:73], R[:, 80:89]], axis=-1).reshape(1, NBOX, 2)

    def plane(lo):
        return R[:, lo:lo + 9].reshape(ROWS, 128)

    y1_pl, x1_pl, y2_pl, x2_pl, s_pl = (plane(96), plane(112), plane(128),
                                        plane(144), plane(160))

    pos, meta = pl.pallas_call(
        _prep_kernel,
        out_shape=[jax.ShapeDtypeStruct((ROWS, 128), jnp.int32),
                   jax.ShapeDtypeStruct((1, 128), jnp.int32)],
    )(s_pl)

    mesh = plsc.VectorSubcoreMesh(core_axis_name="c", subcore_axis_name="s")
    compact = pl.kernel(
        _compact_kernel,
        mesh=mesh,
        out_type=[jax.ShapeDtypeStruct((CCAP,), f32)] * 5,
        scratch_types=[pltpu.VMEM((RT, 128), jnp.int32)]
                      + [pltpu.VMEM((RT, 128), f32)] * 5
                      + [pltpu.SemaphoreType.DMA],
    )
    cs, cy1, cx1, cy2, cx2 = compact(pos, s_pl, y1_pl, x1_pl, y2_pl, x2_pl)

    def cplane(a):
        return a[0:NMSROWS * 128].reshape(NMSROWS, 128)

    b0row = jnp.concatenate([y1_pl[0:1], x1_pl[0:1], y2_pl[0:1], x2_pl[0:1]],
                            axis=0)

    out = pl.pallas_call(
        _nms_kernel,
        out_shape=jax.ShapeDtypeStruct((POST_NMS, 128), f32),
    )(cplane(cy1), cplane(cx1), cplane(cy2), cplane(cx2), cplane(cs),
      meta, b0row)

    proposal_rois = out[:, :4]
    return (conv1, proposal_rois, rpn_bnds, rpn_class)


# speedup vs baseline: 16.2910x; 16.2910x over previous
import functools

import jax
import jax.numpy as jnp
import numpy as np
from jax import lax
from jax.experimental import pallas as pl
from jax.experimental.pallas import tpu as pltpu
from jax.experimental.pallas import tpu_sc as plsc

IMG_H = 64
IMG_W = 64
STRIDE = 16.0
MIN_SIZE = 16.0
PRE_NMS = 6000
POST_NMS = 300
NMS_THRES = 0.7
N_ANCH = 9
NPIX = IMG_H * IMG_W
NBOX = NPIX * N_ANCH
ROWS = NBOX // 128

CROWS = 48
CCAP = CROWS * 128
NMSROWS = 47
ATOTAL = CCAP + NBOX
SC_W = 18
RT = ROWS // SC_W


def _make_anchor_planes():
    xs = np.arange(0.5, IMG_W + 0.5, 1.0, dtype=np.float32)
    ys = np.arange(0.5, IMG_H + 0.5, 1.0, dtype=np.float32)
    scales = [8.0, 16.0, 32.0]
    ratios = [0.5, 1.0, 2.0]
    hw = np.array([[s * np.sqrt(r), s * (1.0 / np.sqrt(r))] for s in scales for r in ratios],
                  dtype=np.float32)
    Y = np.broadcast_to(ys[None, :, None], (IMG_W, IMG_H, N_ANCH))
    X = np.broadcast_to(xs[:, None, None], (IMG_W, IMG_H, N_ANCH))
    H = np.broadcast_to(hw[None, None, :, 0], (IMG_W, IMG_H, N_ANCH))
    W = np.broadcast_to(hw[None, None, :, 1], (IMG_W, IMG_H, N_ANCH))
    anch = np.stack([Y, X, H, W], axis=-1).reshape(-1, 4).astype(np.float32) * STRIDE
    a = anch.reshape(NPIX, N_ANCH, 4)
    ap = np.zeros((NPIX, 64), np.float32)
    for k in range(4):
        ap[:, 16 * k:16 * k + N_ANCH] = a[:, :, k]
    return ap


_AP = _make_anchor_planes()

_NEG = -3.0


def _trunk_kernel(xp_ref, wk_ref, w2_ref, b2_ref, ap_ref,
                  gam_ref, bet_ref, mu_ref, var_ref,
                  conv1_ref, r_ref):
    i = pl.program_id(0)
    acc = jnp.zeros((512, 512), jnp.float32)
    for dy in range(3):
        for dx in range(3):
            sl = xp_ref[pl.ds(8 * i + dy, 8), dx:dx + 64, :]
            sl = sl.reshape(512, 256)
            acc = acc + jnp.dot(sl, wk_ref[3 * dy + dx],
                                preferred_element_type=jnp.float32)
    c = (acc - mu_ref[...]) * jax.lax.rsqrt(var_ref[...] + 1e-5) * gam_ref[...] + bet_ref[...]
    conv1 = jnp.where(c >= 0, c, 0.1 * c)
    conv1_ref[...] = conv1
    h2 = jnp.dot(conv1, w2_ref[...], preferred_element_type=jnp.float32) + b2_ref[...]
    b_y = h2[:, 0:16]
    b_x = h2[:, 16:32]
    b_h = h2[:, 32:48]
    b_w = h2[:, 48:64]
    c0 = h2[:, 64:80]
    c1 = h2[:, 80:96]
    m = jnp.maximum(c0, c1)
    e0 = jnp.exp(c0 - m)
    e1 = jnp.exp(c1 - m)
    den = e0 + e1
    p0 = e0 / den
    p1 = e1 / den
    a_y = ap_ref[:, 0:16]
    a_x = ap_ref[:, 16:32]
    a_h = ap_ref[:, 32:48]
    a_w = ap_ref[:, 48:64]
    r_y = b_y * a_h + a_y
    r_x = b_x * a_w + a_x
    r_h = jnp.exp(b_h) * a_h
    r_w = jnp.exp(b_w) * a_w
    y1 = r_y - r_h / 2.0
    x1 = r_x - r_w / 2.0
    y2 = r_y + r_h / 2.0
    x2 = r_x + r_w / 2.0
    hi = jnp.float32(IMG_H * STRIDE)
    y1 = jnp.clip(y1, 0.0, hi)
    x1 = jnp.clip(x1, 0.0, hi)
    y2 = jnp.clip(y2, 0.0, hi)
    x2 = jnp.clip(x2, 0.0, hi)
    cy = (y1 + y2) / 2.0
    cx = (x1 + x2) / 2.0
    hh = y2 - y1
    ww = x2 - x1
    valid = (hh >= MIN_SIZE) & (ww >= MIN_SIZE)
    s = jnp.where(valid, p1, -1.0)
    Y1 = cy - hh / 2.0
    X1 = cx - ww / 2.0
    Y2 = cy + hh / 2.0
    X2 = cx + ww / 2.0
    zeros = jnp.zeros((512, 80), jnp.float32)
    r_ref[...] = jnp.concatenate(
        [b_y, b_x, b_h, b_w, p0, p1, Y1, X1, Y2, X2, s, zeros], axis=1)


def _prep_kernel(s_ref, pos_ref, meta_ref):
    s = s_ref[...]
    key = jnp.where(s > -0.5, jax.lax.bitcast_convert_type(s, jnp.int32),
                    jnp.int32(-1))
    nvalid = jnp.sum((key >= 0).astype(jnp.int32))

    def bs_body(_, lohi):
        lo, hi = lohi
        mid = lo + (hi - lo + 1) // 2
        cnt = jnp.sum((key >= mid).astype(jnp.int32))
        take = cnt >= PRE_NMS
        return (jnp.where(take, mid, lo), jnp.where(take, hi, mid - 1))

    lo, _ = jax.lax.fori_loop(0, 32, bs_body,
                              (jnp.int32(0), jnp.int32(0x7F800000)))
    teff = jnp.where(nvalid < PRE_NMS, jnp.int32(0), lo)

    keep = key >= teff
    kf = keep.astype(jnp.float32)

    rl = jax.lax.broadcasted_iota(jnp.int32, (128, 128), 0)
    cl = jax.lax.broadcasted_iota(jnp.int32, (128, 128), 1)
    U = (rl <= cl).astype(jnp.float32)
    p = jnp.dot(kf, U, preferred_element_type=jnp.float32)

    rtot = jnp.broadcast_to(p[:, 127:128], (ROWS, 128))
    ri = jax.lax.broadcasted_iota(jnp.int32, (ROWS, ROWS), 0)
    ci = jax.lax.broadcasted_iota(jnp.int32, (ROWS, ROWS), 1)
    tril = (ci < ri).astype(jnp.float32)
    base = jnp.dot(tril, rtot, preferred_element_type=jnp.float32)

    pos = (base + p - kf).astype(jnp.int32)
    pos = jnp.minimum(pos, jnp.int32(CCAP - 1))
    lin = (jax.lax.broadcasted_iota(jnp.int32, (ROWS, 128), 0) * 128
           + jax.lax.broadcasted_iota(jnp.int32, (ROWS, 128), 1))
    pos_ref[...] = jnp.where(keep, pos, jnp.int32(CCAP) + lin)
    total = jnp.sum(keep.astype(jnp.int32))
    meta_ref[...] = jnp.zeros((1, 128), jnp.int32) + total


def _compact_kernel(pos_h, s_h, y1_h, x1_h, y2_h, x2_h,
                    os_h, oy1_h, ox1_h, oy2_h, ox2_h,
                    pv, sv, y1v, x1v, y2v, x2v, sem):
    w = lax.axis_index("c") * 16 + lax.axis_index("s")

    @pl.when(w < SC_W)
    def _():
        base = w * RT
        pltpu.sync_copy(pos_h.at[pl.ds(base, RT)], pv)
        pltpu.sync_copy(s_h.at[pl.ds(base, RT)], sv)
        pltpu.sync_copy(y1_h.at[pl.ds(base, RT)], y1v)
        pltpu.sync_copy(x1_h.at[pl.ds(base, RT)], x1v)
        pltpu.sync_copy(y2_h.at[pl.ds(base, RT)], y2v)
        pltpu.sync_copy(x2_h.at[pl.ds(base, RT)], x2v)
        cps = []
        for src, dst in ((sv, os_h), (y1v, oy1_h), (x1v, ox1_h),
                         (y2v, oy2_h), (x2v, ox2_h)):
            for r in range(RT):
                cps.append(pltpu.async_copy(src.at[r], dst.at[pv.at[r]], sem))
        for c in cps:
            c.wait()


def _nms_kernel(y1_ref, x1_ref, y2_ref, x2_ref, s_ref, cnt_ref, b0_ref,
                out_ref):
    s = s_ref[...]
    y1 = y1_ref[...]
    x1 = x1_ref[...]
    y2 = y2_ref[...]
    x2 = x2_ref[...]
    area = (y2 - y1) * (x2 - x1)
    lin = (jax.lax.broadcasted_iota(jnp.int32, (NMSROWS, 128), 0) * 128
           + jax.lax.broadcasted_iota(jnp.int32, (NMSROWS, 128), 1))
    cio = jax.lax.broadcasted_iota(jnp.int32, (1, 128), 1)

    total = jnp.sum(cnt_ref[...][:, 0:1])
    active = lin < total
    ms0 = jnp.where(active, s, _NEG)

    def pickrow(ref, r, col):
        row = ref[pl.ds(r, 1), :]
        return jnp.sum(jnp.where(cio == col, row, 0.0))

    have = total > 0
    gmax = jnp.max(ms0)
    gidx = jnp.min(jnp.where(ms0 == gmax, lin, jnp.int32(2 ** 30)))
    b0y1 = jnp.sum(jnp.where(cio == 0, b0_ref[0:1, :], 0.0))
    b0x1 = jnp.sum(jnp.where(cio == 0, b0_ref[1:2, :], 0.0))
    b0y2 = jnp.sum(jnp.where(cio == 0, b0_ref[2:3, :], 0.0))
    b0x2 = jnp.sum(jnp.where(cio == 0, b0_ref[3:4, :], 0.0))

    def body(k, ms):
        mmax = jnp.max(ms)
        empty = mmax < -2.0
        idx0 = jnp.min(jnp.where(ms == mmax, lin, jnp.int32(2 ** 30)))
        idx = jnp.where(empty, gidx, idx0)
        r = idx // 128
        col = idx % 128
        y1i = jnp.where(have, pickrow(y1_ref, r, col), b0y1)
        x1i = jnp.where(have, pickrow(x1_ref, r, col), b0x1)
        y2i = jnp.where(have, pickrow(y2_ref, r, col), b0y2)
        x2i = jnp.where(have, pickrow(x2_ref, r, col), b0x2)
        ai = (y2i - y1i) * (x2i - x1i)
        yy1 = jnp.maximum(y1i, y1)
        xx1 = jnp.maximum(x1i, x1)
        yy2 = jnp.minimum(y2i, y2)
        xx2 = jnp.minimum(x2i, x2)
        inter = jnp.maximum(yy2 - yy1, 0.0) * jnp.maximum(xx2 - xx1, 0.0)
        iou = inter / (ai + area - inter + 1e-9)
        ms = jnp.where((iou <= NMS_THRES) & (lin != idx), ms, _NEG)
        outrow = jnp.where(cio == 0, y1i,
                 jnp.where(cio == 1, x1i,
                 jnp.where(cio == 2, y2i,
                 jnp.where(cio == 3, x2i, 0.0))))
        out_ref[pl.ds(k, 1), :] = outrow
        return ms

    jax.lax.fori_loop(0, POST_NMS, body, ms0)


@functools.partial(jax.jit)
def kernel(x, conv_w, bn_gamma, bn_beta, bn_mean, bn_var,
           rpn_bnd_w, rpn_bnd_b, rpn_class_w, rpn_class_b):
    f32 = jnp.float32
    xh = jnp.transpose(x[0], (1, 2, 0))
    xp = jnp.pad(xh, ((1, 1), (1, 1), (0, 0)))
    wk = jnp.transpose(conv_w, (2, 3, 1, 0)).reshape(9, 256, 512)

    wb4 = jnp.transpose(rpn_bnd_w.reshape(N_ANCH, 4, 512), (1, 0, 2))
    wc2 = jnp.transpose(rpn_class_w.reshape(N_ANCH, 2, 512), (1, 0, 2))

    def col_block(wrows):
        return jnp.pad(jnp.transpose(wrows), ((0, 0), (0, 16 - N_ANCH)))

    w2 = jnp.concatenate([col_block(wb4[0]), col_block(wb4[1]),
                          col_block(wb4[2]), col_block(wb4[3]),
                          col_block(wc2[0]), col_block(wc2[1]),
                          jnp.zeros((512, 32), f32)], axis=1)

    def bias_block(brows):
        return jnp.pad(brows, (0, 16 - N_ANCH))

    bb = jnp.transpose(rpn_bnd_b.reshape(N_ANCH, 4))
    cb = jnp.transpose(rpn_class_b.reshape(N_ANCH, 2))
    b2 = jnp.concatenate([bias_block(bb[0]), bias_block(bb[1]),
                          bias_block(bb[2]), bias_block(bb[3]),
                          bias_block(cb[0]), bias_block(cb[1]),
                          jnp.zeros((32,), f32)])[None, :]

    conv1_flat, R = pl.pallas_call(
        _trunk_kernel,
        grid=(8,),
        in_specs=[
            pl.BlockSpec((66, 66, 256), lambda i: (0, 0, 0)),
            pl.BlockSpec((9, 256, 512), lambda i: (0, 0, 0)),
            pl.BlockSpec((512, 128), lambda i: (0, 0)),
            pl.BlockSpec((1, 128), lambda i: (0, 0)),
            pl.BlockSpec((512, 64), lambda i: (i, 0)),
            pl.BlockSpec((1, 512), lambda i: (0, 0)),
            pl.BlockSpec((1, 512), lambda i: (0, 0)),
            pl.BlockSpec((1, 512), lambda i: (0, 0)),
            pl.BlockSpec((1, 512), lambda i: (0, 0)),
        ],
        out_specs=[
            pl.BlockSpec((512, 512), lambda i: (i, 0)),
            pl.BlockSpec((512, 256), lambda i: (i, 0)),
        ],
        out_shape=[
            jax.ShapeDtypeStruct((NPIX, 512), f32),
            jax.ShapeDtypeStruct((NPIX, 256), f32),
        ],
    )(xp, wk, w2, b2, jnp.asarray(_AP),
      bn_gamma[None, :], bn_beta[None, :], bn_mean[None, :], bn_var[None, :])

    conv1 = jnp.transpose(conv1_flat.reshape(IMG_H, IMG_W, 512), (2, 0, 1))[None]
    rpn_bnds = jnp.stack([R[:, 0:9], R[:, 16:25], R[:, 32:41], R[:, 48:57]],
                         axis=-1).reshape(1, NBOX, 4)
    rpn_class = jnp.stack([R[:, 64:73], R[:, 80:89]], axis=-1).reshape(1, NBOX, 2)

    def plane(lo):
        return R[:, lo:lo + 9].reshape(ROWS, 128)

    y1_pl, x1_pl, y2_pl, x2_pl, s_pl = (plane(96), plane(112), plane(128),
                                        plane(144), plane(160))

    pos, meta = pl.pallas_call(
        _prep_kernel,
        out_shape=[jax.ShapeDtypeStruct((ROWS, 128), jnp.int32),
                   jax.ShapeDtypeStruct((1, 128), jnp.int32)],
    )(s_pl)

    mesh = plsc.VectorSubcoreMesh(core_axis_name="c", subcore_axis_name="s")
    compact = pl.kernel(
        _compact_kernel,
        mesh=mesh,
        out_type=[jax.ShapeDtypeStruct((ATOTAL,), f32)] * 5,
        scratch_types=[pltpu.VMEM((RT, 128), jnp.int32)]
                      + [pltpu.VMEM((RT, 128), f32)] * 5
                      + [pltpu.SemaphoreType.DMA],
    )
    cs, cy1, cx1, cy2, cx2 = compact(pos, s_pl, y1_pl, x1_pl, y2_pl, x2_pl)

    def cplane(a):
        return a[0:NMSROWS * 128].reshape(NMSROWS, 128)

    b0row = jnp.concatenate([y1_pl[0:1], x1_pl[0:1], y2_pl[0:1], x2_pl[0:1]],
                            axis=0)

    out = pl.pallas_call(
        _nms_kernel,
        out_shape=jax.ShapeDtypeStruct((POST_NMS, 128), f32),
    )(cplane(cy1), cplane(cx1), cplane(cy2), cplane(cx2), cplane(cs),
      meta, b0row)

    proposal_rois = out[:, :4]
    return (conv1, proposal_rois, rpn_bnds, rpn_class)


# speedup vs baseline: 29.2648x; 1.7964x over previous
import functools

import jax
import jax.numpy as jnp
import numpy as np
from jax import lax
from jax.experimental import pallas as pl
from jax.experimental.pallas import tpu as pltpu
from jax.experimental.pallas import tpu_sc as plsc

IMG_H = 64
IMG_W = 64
STRIDE = 16.0
MIN_SIZE = 16.0
PRE_NMS = 6000
POST_NMS = 300
NMS_THRES = 0.7
N_ANCH = 9
NPIX = IMG_H * IMG_W
NBOX = NPIX * N_ANCH
ROWS = NBOX // 128

CROWS = 48
CCAP = CROWS * 128
NMSROWS = 47
ATOTAL = CCAP + NBOX
SC_W = 18
RT = ROWS // SC_W


def _make_anchor_planes():
    xs = np.arange(0.5, IMG_W + 0.5, 1.0, dtype=np.float32)
    ys = np.arange(0.5, IMG_H + 0.5, 1.0, dtype=np.float32)
    scales = [8.0, 16.0, 32.0]
    ratios = [0.5, 1.0, 2.0]
    hw = np.array([[s * np.sqrt(r), s * (1.0 / np.sqrt(r))] for s in scales for r in ratios],
                  dtype=np.float32)
    Y = np.broadcast_to(ys[None, :, None], (IMG_W, IMG_H, N_ANCH))
    X = np.broadcast_to(xs[:, None, None], (IMG_W, IMG_H, N_ANCH))
    H = np.broadcast_to(hw[None, None, :, 0], (IMG_W, IMG_H, N_ANCH))
    W = np.broadcast_to(hw[None, None, :, 1], (IMG_W, IMG_H, N_ANCH))
    anch = np.stack([Y, X, H, W], axis=-1).reshape(-1, 4).astype(np.float32) * STRIDE
    a = anch.reshape(NPIX, N_ANCH, 4)
    ap = np.zeros((NPIX, 64), np.float32)
    for k in range(4):
        ap[:, 16 * k:16 * k + N_ANCH] = a[:, :, k]
    return ap


_AP = _make_anchor_planes()

_NEG = -3.0


def _trunk_kernel(xp_ref, wk_ref, w2_ref, b2_ref, ap_ref,
                  gam_ref, bet_ref, mu_ref, var_ref,
                  conv1_ref, r_ref):
    i = pl.program_id(0)
    acc = jnp.zeros((512, 512), jnp.float32)
    for dy in range(3):
        for dx in range(3):
            sl = xp_ref[pl.ds(8 * i + dy, 8), dx:dx + 64, :]
            sl = sl.reshape(512, 256)
            acc = acc + jnp.dot(sl, wk_ref[3 * dy + dx],
                                preferred_element_type=jnp.float32)
    c = (acc - mu_ref[...]) * jax.lax.rsqrt(var_ref[...] + 1e-5) * gam_ref[...] + bet_ref[...]
    conv1 = jnp.where(c >= 0, c, 0.1 * c)
    conv1_ref[...] = conv1
    h2 = jnp.dot(conv1, w2_ref[...], preferred_element_type=jnp.float32) + b2_ref[...]
    b_y = h2[:, 0:16]
    b_x = h2[:, 16:32]
    b_h = h2[:, 32:48]
    b_w = h2[:, 48:64]
    c0 = h2[:, 64:80]
    c1 = h2[:, 80:96]
    m = jnp.maximum(c0, c1)
    e0 = jnp.exp(c0 - m)
    e1 = jnp.exp(c1 - m)
    den = e0 + e1
    p0 = e0 / den
    p1 = e1 / den
    a_y = ap_ref[:, 0:16]
    a_x = ap_ref[:, 16:32]
    a_h = ap_ref[:, 32:48]
    a_w = ap_ref[:, 48:64]
    r_y = b_y * a_h + a_y
    r_x = b_x * a_w + a_x
    r_h = jnp.exp(b_h) * a_h
    r_w = jnp.exp(b_w) * a_w
    y1 = r_y - r_h / 2.0
    x1 = r_x - r_w / 2.0
    y2 = r_y + r_h / 2.0
    x2 = r_x + r_w / 2.0
    hi = jnp.float32(IMG_H * STRIDE)
    y1 = jnp.clip(y1, 0.0, hi)
    x1 = jnp.clip(x1, 0.0, hi)
    y2 = jnp.clip(y2, 0.0, hi)
    x2 = jnp.clip(x2, 0.0, hi)
    cy = (y1 + y2) / 2.0
    cx = (x1 + x2) / 2.0
    hh = y2 - y1
    ww = x2 - x1
    valid = (hh >= MIN_SIZE) & (ww >= MIN_SIZE)
    s = jnp.where(valid, p1, -1.0)
    Y1 = cy - hh / 2.0
    X1 = cx - ww / 2.0
    Y2 = cy + hh / 2.0
    X2 = cx + ww / 2.0
    zeros = jnp.zeros((512, 80), jnp.float32)
    r_ref[...] = jnp.concatenate(
        [b_y, b_x, b_h, b_w, p0, p1, Y1, X1, Y2, X2, s, zeros], axis=1)


def _prep_kernel(s_ref, pos_ref, meta_ref):
    s = s_ref[...]
    key = jnp.where(s > -0.5, jax.lax.bitcast_convert_type(s, jnp.int32),
                    jnp.int32(-1))
    nvalid = jnp.sum((key >= 0).astype(jnp.int32))

    def bs_body(_, lohi):
        lo, hi = lohi
        mid = lo + (hi - lo + 1) // 2
        cnt = jnp.sum((key >= mid).astype(jnp.int32))
        take = cnt >= PRE_NMS
        return (jnp.where(take, mid, lo), jnp.where(take, hi, mid - 1))

    lo, _ = jax.lax.fori_loop(0, 32, bs_body,
                              (jnp.int32(0), jnp.int32(0x7F800000)))
    teff = jnp.where(nvalid < PRE_NMS, jnp.int32(0), lo)

    keep = key >= teff
    kf = keep.astype(jnp.float32)

    rl = jax.lax.broadcasted_iota(jnp.int32, (128, 128), 0)
    cl = jax.lax.broadcasted_iota(jnp.int32, (128, 128), 1)
    U = (rl <= cl).astype(jnp.float32)
    p = jnp.dot(kf, U, preferred_element_type=jnp.float32)

    rtot = jnp.broadcast_to(p[:, 127:128], (ROWS, 128))
    ri = jax.lax.broadcasted_iota(jnp.int32, (ROWS, ROWS), 0)
    ci = jax.lax.broadcasted_iota(jnp.int32, (ROWS, ROWS), 1)
    tril = (ci < ri).astype(jnp.float32)
    base = jnp.dot(tril, rtot, preferred_element_type=jnp.float32)

    pos = (base + p - kf).astype(jnp.int32)
    pos = jnp.minimum(pos, jnp.int32(CCAP - 1))
    lin = (jax.lax.broadcasted_iota(jnp.int32, (ROWS, 128), 0) * 128
           + jax.lax.broadcasted_iota(jnp.int32, (ROWS, 128), 1))
    pos_ref[...] = jnp.where(keep, pos, jnp.int32(CCAP) + lin)
    total = jnp.sum(keep.astype(jnp.int32))
    meta_ref[...] = jnp.zeros((1, 128), jnp.int32) + total


CH = CCAP // 6


def _scatidx_kernel(pos_h, lin_h, src_h, pv, lv, sem):
    w = lax.axis_index("c") * 16 + lax.axis_index("s")

    @pl.when(w < SC_W)
    def _():
        base = w * RT
        pltpu.sync_copy(pos_h.at[pl.ds(base, RT)], pv)
        pltpu.sync_copy(lin_h.at[pl.ds(base, RT)], lv)
        cps = [pltpu.async_copy(lv.at[r], src_h.at[pv.at[r]], sem)
               for r in range(RT)]
        for c in cps:
            c.wait()


def _gather_kernel(src_h, s_h, y1_h, x1_h, y2_h, x2_h,
                   os_h, oy1_h, ox1_h, oy2_h, ox2_h, iv, gv, sem):
    w = lax.axis_index("c") * 16 + lax.axis_index("s")
    planes = ((s_h, os_h), (y1_h, oy1_h), (x1_h, ox1_h),
              (y2_h, oy2_h), (x2_h, ox2_h))
    for pi, (ih, oh) in enumerate(planes):

        @pl.when((w >= 6 * pi) & (w < 6 * pi + 6))
        def _(ih=ih, oh=oh, pi=pi):
            off = (w - 6 * pi) * CH
            pltpu.sync_copy(src_h.at[pl.ds(off, CH)], iv)
            for k in range(CH // 16):
                t = iv[pl.ds(16 * k, 16)]
                iv[pl.ds(16 * k, 16)] = jnp.minimum(
                    jnp.maximum(t, jnp.int32(0)), jnp.int32(NBOX - 1))
            pltpu.async_copy(ih.at[iv], gv, sem).wait()
            pltpu.sync_copy(gv, oh.at[pl.ds(off, CH)])


def _nms_kernel(y1_ref, x1_ref, y2_ref, x2_ref, s_ref, cnt_ref, b0_ref,
                out_ref):
    s = s_ref[...]
    y1 = y1_ref[...]
    x1 = x1_ref[...]
    y2 = y2_ref[...]
    x2 = x2_ref[...]
    area = (y2 - y1) * (x2 - x1)
    lin = (jax.lax.broadcasted_iota(jnp.int32, (NMSROWS, 128), 0) * 128
           + jax.lax.broadcasted_iota(jnp.int32, (NMSROWS, 128), 1))
    cio = jax.lax.broadcasted_iota(jnp.int32, (1, 128), 1)

    total = jnp.sum(cnt_ref[...][:, 0:1])
    active = lin < total
    ms0 = jnp.where(active, s, _NEG)

    def pickrow(ref, r, col):
        row = ref[pl.ds(r, 1), :]
        return jnp.sum(jnp.where(cio == col, row, 0.0))

    have = total > 0
    gmax = jnp.max(ms0)
    gidx = jnp.min(jnp.where(ms0 == gmax, lin, jnp.int32(2 ** 30)))
    b0y1 = jnp.sum(jnp.where(cio == 0, b0_ref[0:1, :], 0.0))
    b0x1 = jnp.sum(jnp.where(cio == 0, b0_ref[1:2, :], 0.0))
    b0y2 = jnp.sum(jnp.where(cio == 0, b0_ref[2:3, :], 0.0))
    b0x2 = jnp.sum(jnp.where(cio == 0, b0_ref[3:4, :], 0.0))

    def body(k, ms):
        mmax = jnp.max(ms)
        empty = mmax < -2.0
        idx0 = jnp.min(jnp.where(ms == mmax, lin, jnp.int32(2 ** 30)))
        idx = jnp.where(empty, gidx, idx0)
        r = idx // 128
        col = idx % 128
        y1i = jnp.where(have, pickrow(y1_ref, r, col), b0y1)
        x1i = jnp.where(have, pickrow(x1_ref, r, col), b0x1)
        y2i = jnp.where(have, pickrow(y2_ref, r, col), b0y2)
        x2i = jnp.where(have, pickrow(x2_ref, r, col), b0x2)
        ai = (y2i - y1i) * (x2i - x1i)
        yy1 = jnp.maximum(y1i, y1)
        xx1 = jnp.maximum(x1i, x1)
        yy2 = jnp.minimum(y2i, y2)
        xx2 = jnp.minimum(x2i, x2)
        inter = jnp.maximum(yy2 - yy1, 0.0) * jnp.maximum(xx2 - xx1, 0.0)
        iou = inter / (ai + area - inter + 1e-9)
        ms = jnp.where((iou <= NMS_THRES) & (lin != idx), ms, _NEG)
        outrow = jnp.where(cio == 0, y1i,
                 jnp.where(cio == 1, x1i,
                 jnp.where(cio == 2, y2i,
                 jnp.where(cio == 3, x2i, 0.0))))
        out_ref[pl.ds(k, 1), :] = outrow
        return ms

    jax.lax.fori_loop(0, POST_NMS, body, ms0)


@functools.partial(jax.jit)
def kernel(x, conv_w, bn_gamma, bn_beta, bn_mean, bn_var,
           rpn_bnd_w, rpn_bnd_b, rpn_class_w, rpn_class_b):
    f32 = jnp.float32
    xh = jnp.transpose(x[0], (1, 2, 0))
    xp = jnp.pad(xh, ((1, 1), (1, 1), (0, 0)))
    wk = jnp.transpose(conv_w, (2, 3, 1, 0)).reshape(9, 256, 512)

    wb4 = jnp.transpose(rpn_bnd_w.reshape(N_ANCH, 4, 512), (1, 0, 2))
    wc2 = jnp.transpose(rpn_class_w.reshape(N_ANCH, 2, 512), (1, 0, 2))

    def col_block(wrows):
        return jnp.pad(jnp.transpose(wrows), ((0, 0), (0, 16 - N_ANCH)))

    w2 = jnp.concatenate([col_block(wb4[0]), col_block(wb4[1]),
                          col_block(wb4[2]), col_block(wb4[3]),
                          col_block(wc2[0]), col_block(wc2[1]),
                          jnp.zeros((512, 32), f32)], axis=1)

    def bias_block(brows):
        return jnp.pad(brows, (0, 16 - N_ANCH))

    bb = jnp.transpose(rpn_bnd_b.reshape(N_ANCH, 4))
    cb = jnp.transpose(rpn_class_b.reshape(N_ANCH, 2))
    b2 = jnp.concatenate([bias_block(bb[0]), bias_block(bb[1]),
                          bias_block(bb[2]), bias_block(bb[3]),
                          bias_block(cb[0]), bias_block(cb[1]),
                          jnp.zeros((32,), f32)])[None, :]

    conv1_flat, R = pl.pallas_call(
        _trunk_kernel,
        grid=(8,),
        in_specs=[
            pl.BlockSpec((66, 66, 256), lambda i: (0, 0, 0)),
            pl.BlockSpec((9, 256, 512), lambda i: (0, 0, 0)),
            pl.BlockSpec((512, 128), lambda i: (0, 0)),
            pl.BlockSpec((1, 128), lambda i: (0, 0)),
            pl.BlockSpec((512, 64), lambda i: (i, 0)),
            pl.BlockSpec((1, 512), lambda i: (0, 0)),
            pl.BlockSpec((1, 512), lambda i: (0, 0)),
            pl.BlockSpec((1, 512), lambda i: (0, 0)),
            pl.BlockSpec((1, 512), lambda i: (0, 0)),
        ],
        out_specs=[
            pl.BlockSpec((512, 512), lambda i: (i, 0)),
            pl.BlockSpec((512, 256), lambda i: (i, 0)),
        ],
        out_shape=[
            jax.ShapeDtypeStruct((NPIX, 512), f32),
            jax.ShapeDtypeStruct((NPIX, 256), f32),
        ],
    )(xp, wk, w2, b2, jnp.asarray(_AP),
      bn_gamma[None, :], bn_beta[None, :], bn_mean[None, :], bn_var[None, :])

    conv1 = jnp.transpose(conv1_flat.reshape(IMG_H, IMG_W, 512), (2, 0, 1))[None]
    rpn_bnds = jnp.stack([R[:, 0:9], R[:, 16:25], R[:, 32:41], R[:, 48:57]],
                         axis=-1).reshape(1, NBOX, 4)
    rpn_class = jnp.stack([R[:, 64:73], R[:, 80:89]], axis=-1).reshape(1, NBOX, 2)

    def plane(lo):
        return R[:, lo:lo + 9].reshape(ROWS, 128)

    y1_pl, x1_pl, y2_pl, x2_pl, s_pl = (plane(96), plane(112), plane(128),
                                        plane(144), plane(160))

    pos, meta = pl.pallas_call(
        _prep_kernel,
        out_shape=[jax.ShapeDtypeStruct((ROWS, 128), jnp.int32),
                   jax.ShapeDtypeStruct((1, 128), jnp.int32)],
    )(s_pl)

    mesh = plsc.VectorSubcoreMesh(core_axis_name="c", subcore_axis_name="s")
    scat = pl.kernel(
        _scatidx_kernel,
        mesh=mesh,
        out_type=[jax.ShapeDtypeStruct((ATOTAL,), jnp.int32)],
        scratch_types=[pltpu.VMEM((RT, 128), jnp.int32),
                       pltpu.VMEM((RT, 128), jnp.int32),
                       pltpu.SemaphoreType.DMA],
    )
    lin_pl = jnp.arange(NBOX, dtype=jnp.int32).reshape(ROWS, 128)
    (src,) = scat(pos, lin_pl)

    gath = pl.kernel(
        _gather_kernel,
        mesh=mesh,
        out_type=[jax.ShapeDtypeStruct((CCAP,), f32)] * 5,
        scratch_types=[pltpu.VMEM((CH,), jnp.int32),
                       pltpu.VMEM((CH,), f32),
                       pltpu.SemaphoreType.DMA],
    )
    cs, cy1, cx1, cy2, cx2 = gath(src[0:CCAP], s_pl.reshape(NBOX),
                                  y1_pl.reshape(NBOX), x1_pl.reshape(NBOX),
                                  y2_pl.reshape(NBOX), x2_pl.reshape(NBOX))

    def cplane(a):
        return a[0:NMSROWS * 128].reshape(NMSROWS, 128)

    b0row = jnp.concatenate([y1_pl[0:1], x1_pl[0:1], y2_pl[0:1], x2_pl[0:1]],
                            axis=0)

    out = pl.pallas_call(
        _nms_kernel,
        out_shape=jax.ShapeDtypeStruct((POST_NMS, 128), f32),
    )(cplane(cy1), cplane(cx1), cplane(cy2), cplane(cx2), cplane(cs),
      meta, b0row)

    proposal_rois = out[:, :4]
    return (conv1, proposal_rois, rpn_bnds, rpn_class)
